# Initial kernel scaffold; baseline (speedup 1.0000x reference)
import jax
import jax.numpy as jnp
from jax.experimental import pallas as pl


def kernel(x, base_embed, distance_embed, W, b):
    raise NotImplementedError("write your pallas kernel here")


# speedup vs baseline: 1.5057x; 1.5057x over previous
import functools

import jax
import jax.numpy as jnp
from jax import lax
from jax.experimental import pallas as pl
from jax.experimental.pallas import tpu as pltpu
from jax.experimental.pallas import tpu_sc as plsc

NUM_DIST = 3
NUM_BASE = 5
EMBED = 5
EDGE_EMBED = 128
TBL = 16


def _table_body(base_ref, dist_ref, w_ref, b_ref, out_ref):
    v_b = lax.broadcasted_iota(jnp.int32, (TBL, NUM_BASE), 0)
    c_b = lax.broadcasted_iota(jnp.int32, (TBL, NUM_BASE), 1)
    v_d = lax.broadcasted_iota(jnp.int32, (TBL, NUM_DIST), 0)
    c_d = lax.broadcasted_iota(jnp.int32, (TBL, NUM_DIST), 1)
    bi = jnp.minimum(v_b // NUM_DIST, NUM_BASE - 1)
    di = v_d % NUM_DIST
    onehot_b = (c_b == bi).astype(jnp.float32)
    onehot_d = (c_d == di).astype(jnp.float32)
    be = jnp.dot(onehot_b, base_ref[...], preferred_element_type=jnp.float32)
    de = jnp.dot(onehot_d, dist_ref[...], preferred_element_type=jnp.float32)
    h = jax.nn.relu(jnp.concatenate([be, de], axis=1))
    out = lax.dot_general(h, w_ref[...], (((1,), (1,)), ((), ())),
                          preferred_element_type=jnp.float32)
    out_ref[...] = out + b_ref[...][None, :]


def _make_table(base_embed, distance_embed, W, b):
    return pl.pallas_call(
        _table_body,
        out_shape=jax.ShapeDtypeStruct((TBL, EDGE_EMBED), jnp.float32),
    )(base_embed, distance_embed, W, b)


def _make_gather(E, C):
    info = plsc.get_sparse_core_info()
    NC, NS = info.num_cores, info.num_subcores
    NW = NC * NS
    n_chunks = E // C
    per_worker = -(-n_chunks // NW)

    @functools.partial(
        pl.kernel,
        mesh=plsc.VectorSubcoreMesh(core_axis_name="c", subcore_axis_name="s"),
        out_type=jax.ShapeDtypeStruct((E, EDGE_EMBED), jnp.float32),
        scratch_types=[
            pltpu.VMEM((C,), jnp.int32),
            pltpu.VMEM((C, EDGE_EMBED), jnp.float32),
            pltpu.SemaphoreType.DMA,
        ],
    )
    def gather(table_hbm, x_hbm, out_hbm, idx_v, rows_v, sem):
        wid = lax.axis_index("s") * NC + lax.axis_index("c")

        def body(k, _):
            cid = k * NW + wid

            @pl.when(cid < n_chunks)
            def _():
                base = cid * C
                pltpu.sync_copy(x_hbm.at[pl.ds(base, C)], idx_v)
                pltpu.async_copy(table_hbm.at[idx_v], rows_v, sem).wait()
                pltpu.sync_copy(rows_v, out_hbm.at[pl.ds(base, C)])

            return ()

        lax.fori_loop(0, per_worker, body, (), unroll=False)

    return gather


def kernel(x, base_embed, distance_embed, W, b):
    table = _make_table(base_embed, distance_embed, W, b)
    E = x.shape[0]
    gather = _make_gather(E, 512)
    return gather(table, x.astype(jnp.int32))


# speedup vs baseline: 1.5069x; 1.0008x over previous
import functools

import jax
import jax.numpy as jnp
from jax import lax
from jax.experimental import pallas as pl
from jax.experimental.pallas import tpu as pltpu
from jax.experimental.pallas import tpu_sc as plsc

NUM_DIST = 3
NUM_BASE = 5
EMBED = 5
EDGE_EMBED = 128
TBL = 16


def _table_body(base_ref, dist_ref, w_ref, b_ref, out_ref):
    v_b = lax.broadcasted_iota(jnp.int32, (TBL, NUM_BASE), 0)
    c_b = lax.broadcasted_iota(jnp.int32, (TBL, NUM_BASE), 1)
    v_d = lax.broadcasted_iota(jnp.int32, (TBL, NUM_DIST), 0)
    c_d = lax.broadcasted_iota(jnp.int32, (TBL, NUM_DIST), 1)
    bi = jnp.minimum(v_b // NUM_DIST, NUM_BASE - 1)
    di = v_d % NUM_DIST
    onehot_b = (c_b == bi).astype(jnp.float32)
    onehot_d = (c_d == di).astype(jnp.float32)
    be = jnp.dot(onehot_b, base_ref[...], preferred_element_type=jnp.float32)
    de = jnp.dot(onehot_d, dist_ref[...], preferred_element_type=jnp.float32)
    h = jax.nn.relu(jnp.concatenate([be, de], axis=1))
    out = lax.dot_general(h, w_ref[...], (((1,), (1,)), ((), ())),
                          preferred_element_type=jnp.float32)
    out_ref[...] = out + b_ref[...][None, :]


def _make_table(base_embed, distance_embed, W, b):
    return pl.pallas_call(
        _table_body,
        out_shape=jax.ShapeDtypeStruct((TBL, EDGE_EMBED), jnp.float32),
    )(base_embed, distance_embed, W, b)


def _make_gather(E, C=400, NB=2):
    info = plsc.get_sparse_core_info()
    NC, NS = info.num_cores, info.num_subcores
    NW = NC * NS
    n_chunks = E // C
    assert n_chunks * C == E and n_chunks % NW == 0 and (C * 4) % 64 == 0
    per_worker = n_chunks // NW

    @functools.partial(
        pl.kernel,
        mesh=plsc.VectorSubcoreMesh(core_axis_name="c", subcore_axis_name="s"),
        out_type=jax.ShapeDtypeStruct((E, EDGE_EMBED), jnp.float32),
        scratch_types=[
            pltpu.VMEM((C,), jnp.int32),
            pltpu.VMEM((C,), jnp.int32),
            pltpu.VMEM((NB, C, EDGE_EMBED), jnp.float32),
            pltpu.SemaphoreType.DMA,
            pltpu.SemaphoreType.DMA,
            pltpu.SemaphoreType.DMA,
        ],
    )
    def gather(table_hbm, x_hbm, out_hbm, idx0, idx1, rows_v, gsem, ssem0, ssem1):
        idxs = (idx0, idx1)
        wid = lax.axis_index("s") * NC + lax.axis_index("c")
        ssems = (ssem0, ssem1)

        def drain_store(b):
            pltpu.make_async_copy(
                rows_v.at[b], out_hbm.at[pl.ds(0, C)], ssems[b]).wait()

        def do_chunk(k, b, drain):
            base = (k * NW + wid) * C
            if drain:
                drain_store(b)
            pltpu.sync_copy(x_hbm.at[pl.ds(base, C)], idxs[b])
            pltpu.async_copy(table_hbm.at[idxs[b]], rows_v.at[b], gsem).wait()
            pltpu.async_copy(rows_v.at[b], out_hbm.at[pl.ds(base, C)], ssems[b])

        head = min(NB, per_worker)
        tail = (per_worker - head) % NB
        main = (per_worker - head - tail) // NB
        for b in range(head):
            do_chunk(b, b, drain=False)

        def body(g, _):
            for b in range(NB):
                do_chunk(head + g * NB + b, b, drain=True)
            return ()

        lax.fori_loop(0, main, body, (), unroll=False)
        for t in range(tail):
            k = per_worker - tail + t
            do_chunk(k, k % NB, drain=True)
        for b in range(head):
            drain_store(b)

    return gather


def kernel(x, base_embed, distance_embed, W, b):
    table = _make_table(base_embed, distance_embed, W, b)
    E = x.shape[0]
    gather = _make_gather(E)
    return gather(table, x.astype(jnp.int32))


# speedup vs baseline: 6.1756x; 4.0982x over previous
import functools

import jax
import jax.numpy as jnp
from jax import lax
from jax.experimental import pallas as pl
from jax.experimental.pallas import tpu as pltpu
from jax.experimental.pallas import tpu_sc as plsc

NUM_DIST = 3
NUM_BASE = 5
EMBED = 5
EDGE_EMBED = 128
TBL = 16


def _table_body(base_ref, dist_ref, w_ref, b_ref, out_ref):
    v_b = lax.broadcasted_iota(jnp.int32, (TBL, NUM_BASE), 0)
    c_b = lax.broadcasted_iota(jnp.int32, (TBL, NUM_BASE), 1)
    v_d = lax.broadcasted_iota(jnp.int32, (TBL, NUM_DIST), 0)
    c_d = lax.broadcasted_iota(jnp.int32, (TBL, NUM_DIST), 1)
    bi = jnp.minimum(v_b // NUM_DIST, NUM_BASE - 1)
    di = v_d % NUM_DIST
    onehot_b = (c_b == bi).astype(jnp.float32)
    onehot_d = (c_d == di).astype(jnp.float32)
    be = jnp.dot(onehot_b, base_ref[...], preferred_element_type=jnp.float32)
    de = jnp.dot(onehot_d, dist_ref[...], preferred_element_type=jnp.float32)
    h = jax.nn.relu(jnp.concatenate([be, de], axis=1))
    out = lax.dot_general(h, w_ref[...], (((1,), (1,)), ((), ())),
                          preferred_element_type=jnp.float32)
    out = out + b_ref[...][None, :]
    out_ref[...] = jnp.broadcast_to(out[None], (NREP, TBL, EDGE_EMBED))


NREP = 32


def _make_table(base_embed, distance_embed, W, b):
    return pl.pallas_call(
        _table_body,
        out_shape=jax.ShapeDtypeStruct((NREP, TBL, EDGE_EMBED), jnp.float32),
    )(base_embed, distance_embed, W, b)


def _make_gather(E, C=400, NB=2):
    info = plsc.get_sparse_core_info()
    NC, NS = info.num_cores, info.num_subcores
    NW = NC * NS
    n_chunks = E // C
    assert n_chunks * C == E and n_chunks % NW == 0 and (C * 4) % 64 == 0
    per_worker = n_chunks // NW

    @functools.partial(
        pl.kernel,
        mesh=plsc.VectorSubcoreMesh(core_axis_name="c", subcore_axis_name="s"),
        out_type=jax.ShapeDtypeStruct((E, EDGE_EMBED), jnp.float32),
        scratch_types=[
            pltpu.VMEM((C,), jnp.int32),
            pltpu.VMEM((C,), jnp.int32),
            pltpu.VMEM((NB, C, EDGE_EMBED), jnp.float32),
            pltpu.SemaphoreType.DMA,
            pltpu.SemaphoreType.DMA,
            pltpu.SemaphoreType.DMA,
        ],
    )
    def gather(table_hbm, x_hbm, out_hbm, idx0, idx1, rows_v, gsem, ssem0, ssem1):
        idxs = (idx0, idx1)
        wid = lax.axis_index("s") * NC + lax.axis_index("c")
        ssems = (ssem0, ssem1)
        row_off = wid * TBL

        def drain_store(b):
            pltpu.make_async_copy(
                rows_v.at[b], out_hbm.at[pl.ds(0, C)], ssems[b]).wait()

        def do_chunk(k, b, drain):
            base = (k * NW + wid) * C
            if drain:
                drain_store(b)
            pltpu.sync_copy(x_hbm.at[pl.ds(base, C)], idxs[b])
            for i in range(C // 16):
                sl = pl.ds(i * 16, 16)
                idxs[b][sl] = idxs[b][sl] + row_off
            pltpu.async_copy(table_hbm.at[idxs[b]], rows_v.at[b], gsem).wait()
            pltpu.async_copy(rows_v.at[b], out_hbm.at[pl.ds(base, C)], ssems[b])

        head = min(NB, per_worker)
        tail = (per_worker - head) % NB
        main = (per_worker - head - tail) // NB
        for b in range(head):
            do_chunk(b, b, drain=False)

        def body(g, _):
            for b in range(NB):
                do_chunk(head + g * NB + b, b, drain=True)
            return ()

        lax.fori_loop(0, main, body, (), unroll=False)
        for t in range(tail):
            k = per_worker - tail + t
            do_chunk(k, k % NB, drain=True)
        for b in range(head):
            drain_store(b)

    return gather


def kernel(x, base_embed, distance_embed, W, b):
    table = _make_table(base_embed, distance_embed, W, b)
    table = table.reshape(NREP * TBL, EDGE_EMBED)
    E = x.shape[0]
    gather = _make_gather(E)
    return gather(table, x.astype(jnp.int32))


# speedup vs baseline: 21.5030x; 3.4819x over previous
import functools

import jax
import jax.numpy as jnp
from jax import lax
from jax.experimental import pallas as pl
from jax.experimental.pallas import tpu as pltpu
from jax.experimental.pallas import tpu_sc as plsc

NUM_DIST = 3
NUM_BASE = 5
EMBED = 5
EDGE_EMBED = 128
TBL = 16


def _table_body(base_ref, dist_ref, w_ref, b_ref, out_ref):
    v_b = lax.broadcasted_iota(jnp.int32, (TBL, NUM_BASE), 0)
    c_b = lax.broadcasted_iota(jnp.int32, (TBL, NUM_BASE), 1)
    v_d = lax.broadcasted_iota(jnp.int32, (TBL, NUM_DIST), 0)
    c_d = lax.broadcasted_iota(jnp.int32, (TBL, NUM_DIST), 1)
    bi = jnp.minimum(v_b // NUM_DIST, NUM_BASE - 1)
    di = v_d % NUM_DIST
    onehot_b = (c_b == bi).astype(jnp.float32)
    onehot_d = (c_d == di).astype(jnp.float32)
    be = jnp.dot(onehot_b, base_ref[...], preferred_element_type=jnp.float32)
    de = jnp.dot(onehot_d, dist_ref[...], preferred_element_type=jnp.float32)
    h = jax.nn.relu(jnp.concatenate([be, de], axis=1))
    out = lax.dot_general(h, w_ref[...], (((1,), (1,)), ((), ())),
                          preferred_element_type=jnp.float32)
    out = out + b_ref[...][None, :]
    out_ref[...] = jnp.broadcast_to(out[None], (NREP, TBL, EDGE_EMBED))


NREP = 32


def _make_table(base_embed, distance_embed, W, b):
    return pl.pallas_call(
        _table_body,
        out_shape=jax.ShapeDtypeStruct((NREP, TBL, EDGE_EMBED), jnp.float32),
    )(base_embed, distance_embed, W, b)


def _make_gather(E, C=400, NB=2):
    info = plsc.get_sparse_core_info()
    NC, NS = info.num_cores, info.num_subcores
    NW = NC * NS
    n_chunks = E // C
    assert n_chunks * C == E and n_chunks % NW == 0 and (C * 4) % 64 == 0
    per_worker = n_chunks // NW

    @functools.partial(
        pl.kernel,
        mesh=plsc.VectorSubcoreMesh(core_axis_name="c", subcore_axis_name="s"),
        out_type=jax.ShapeDtypeStruct((E, EDGE_EMBED), jnp.float32),
        scratch_types=[
            pltpu.VMEM((C,), jnp.int32),
            pltpu.VMEM((C,), jnp.int32),
            pltpu.VMEM((NB, C, EDGE_EMBED), jnp.float32),
            pltpu.VMEM_SHARED((NREP * TBL, EDGE_EMBED), jnp.float32),
            pltpu.SemaphoreType.DMA,
            pltpu.SemaphoreType.DMA,
            pltpu.SemaphoreType.DMA,
        ],
    )
    def gather(table_hbm, x_hbm, out_hbm, idx0, idx1, rows_v, table_sh,
               gsem, ssem0, ssem1):
        idxs = (idx0, idx1)
        sid = lax.axis_index("s")
        wid = sid * NC + lax.axis_index("c")
        ssems = (ssem0, ssem1)
        row_off = wid * TBL

        @pl.when(sid == 0)
        def _():
            pltpu.sync_copy(table_hbm, table_sh)

        plsc.subcore_barrier()

        def drain_store(b):
            pltpu.make_async_copy(
                rows_v.at[b], out_hbm.at[pl.ds(0, C)], ssems[b]).wait()

        def do_chunk(k, b, drain):
            base = (k * NW + wid) * C
            if drain:
                drain_store(b)
            pltpu.sync_copy(x_hbm.at[pl.ds(base, C)], idxs[b])
            for i in range(C // 16):
                sl = pl.ds(i * 16, 16)
                idxs[b][sl] = idxs[b][sl] + row_off
            pltpu.async_copy(table_sh.at[idxs[b]], rows_v.at[b], gsem).wait()
            pltpu.async_copy(rows_v.at[b], out_hbm.at[pl.ds(base, C)], ssems[b])

        head = min(NB, per_worker)
        tail = (per_worker - head) % NB
        main = (per_worker - head - tail) // NB
        for b in range(head):
            do_chunk(b, b, drain=False)

        def body(g, _):
            for b in range(NB):
                do_chunk(head + g * NB + b, b, drain=True)
            return ()

        lax.fori_loop(0, main, body, (), unroll=False)
        for t in range(tail):
            k = per_worker - tail + t
            do_chunk(k, k % NB, drain=True)
        for b in range(head):
            drain_store(b)

    return gather


def kernel(x, base_embed, distance_embed, W, b):
    table = _make_table(base_embed, distance_embed, W, b)
    table = table.reshape(NREP * TBL, EDGE_EMBED)
    E = x.shape[0]
    gather = _make_gather(E)
    return gather(table, x.astype(jnp.int32))


# speedup vs baseline: 24.5362x; 1.1411x over previous
import functools

import jax
import jax.numpy as jnp
from jax import lax
from jax.experimental import pallas as pl
from jax.experimental.pallas import tpu as pltpu
from jax.experimental.pallas import tpu_sc as plsc

NUM_DIST = 3
NUM_BASE = 5
EMBED = 5
EDGE_EMBED = 128
TBL = 16


def _table_body(base_ref, dist_ref, w_ref, b_ref, out_ref):
    v_b = lax.broadcasted_iota(jnp.int32, (TBL, NUM_BASE), 0)
    c_b = lax.broadcasted_iota(jnp.int32, (TBL, NUM_BASE), 1)
    v_d = lax.broadcasted_iota(jnp.int32, (TBL, NUM_DIST), 0)
    c_d = lax.broadcasted_iota(jnp.int32, (TBL, NUM_DIST), 1)
    bi = jnp.minimum(v_b // NUM_DIST, NUM_BASE - 1)
    di = v_d % NUM_DIST
    onehot_b = (c_b == bi).astype(jnp.float32)
    onehot_d = (c_d == di).astype(jnp.float32)
    be = jnp.dot(onehot_b, base_ref[...], preferred_element_type=jnp.float32)
    de = jnp.dot(onehot_d, dist_ref[...], preferred_element_type=jnp.float32)
    h = jax.nn.relu(jnp.concatenate([be, de], axis=1))
    out = lax.dot_general(h, w_ref[...], (((1,), (1,)), ((), ())),
                          preferred_element_type=jnp.float32)
    out = out + b_ref[...][None, :]
    out_ref[...] = jnp.broadcast_to(out[None], (NREP, TBL, EDGE_EMBED))


NREP = 32


def _make_table(base_embed, distance_embed, W, b):
    return pl.pallas_call(
        _table_body,
        out_shape=jax.ShapeDtypeStruct((NREP, TBL, EDGE_EMBED), jnp.float32),
    )(base_embed, distance_embed, W, b)


def _make_gather(E, C=400, NB=2):
    info = plsc.get_sparse_core_info()
    NC, NS = info.num_cores, info.num_subcores
    NW = NC * NS
    n_chunks = E // C
    assert n_chunks * C == E and n_chunks % NW == 0 and (C * 4) % 64 == 0
    per_worker = n_chunks // NW

    @functools.partial(
        pl.kernel,
        mesh=plsc.VectorSubcoreMesh(core_axis_name="c", subcore_axis_name="s"),
        out_type=jax.ShapeDtypeStruct((E, EDGE_EMBED), jnp.float32),
        scratch_types=[
            pltpu.VMEM((C,), jnp.int32),
            pltpu.VMEM((C,), jnp.int32),
            pltpu.VMEM((NB, C, EDGE_EMBED), jnp.float32),
            pltpu.VMEM_SHARED((NREP * TBL, EDGE_EMBED), jnp.float32),
            pltpu.SemaphoreType.DMA,
            pltpu.SemaphoreType.DMA,
            pltpu.SemaphoreType.DMA,
            pltpu.SemaphoreType.DMA,
            pltpu.SemaphoreType.DMA,
        ],
    )
    def gather(table_hbm, x_hbm, out_hbm, idx0, idx1, rows_v, table_sh,
               gsem, ssem0, ssem1, isem0, isem1):
        idxs = (idx0, idx1)
        isems = (isem0, isem1)
        ssems = (ssem0, ssem1)
        sid = lax.axis_index("s")
        wid = sid * NC + lax.axis_index("c")
        row_off = wid * TBL

        @pl.when(sid == 0)
        def _():
            pltpu.sync_copy(table_hbm, table_sh)

        plsc.subcore_barrier()

        def idx_load(k, b):
            base = (k * NW + wid) * C
            pltpu.async_copy(x_hbm.at[pl.ds(base, C)], idxs[b], isems[b])

        def drain_store(b):
            pltpu.make_async_copy(
                rows_v.at[b], out_hbm.at[pl.ds(0, C)], ssems[b]).wait()

        def do_chunk(k, b, drain):
            base = (k * NW + wid) * C
            if drain:
                drain_store(b)
            pltpu.make_async_copy(
                x_hbm.at[pl.ds(0, C)], idxs[b], isems[b]).wait()
            for i in range(C // 16):
                sl = pl.ds(i * 16, 16)
                idxs[b][sl] = idxs[b][sl] + row_off
            pltpu.async_copy(table_sh.at[idxs[b]], rows_v.at[b], gsem).wait()
            nk = k + NB
            if isinstance(k, int):
                if nk < per_worker:
                    idx_load(nk, b)
            else:
                @pl.when(nk < per_worker)
                def _():
                    idx_load(nk, b)
            pltpu.async_copy(rows_v.at[b], out_hbm.at[pl.ds(base, C)], ssems[b])

        head = min(NB, per_worker)
        tail = (per_worker - head) % NB
        main = (per_worker - head - tail) // NB
        for b in range(head):
            idx_load(b, b)
        for b in range(head):
            do_chunk(b, b, drain=False)

        def body(g, _):
            for b in range(NB):
                do_chunk(head + g * NB + b, b, drain=True)
            return ()

        lax.fori_loop(0, main, body, (), unroll=False)
        for t in range(tail):
            k = per_worker - tail + t
            do_chunk(k, k % NB, drain=True)
        for b in range(head):
            drain_store(b)

    return gather


def kernel(x, base_embed, distance_embed, W, b):
    table = _make_table(base_embed, distance_embed, W, b)
    table = table.reshape(NREP * TBL, EDGE_EMBED)
    E = x.shape[0]
    gather = _make_gather(E)
    return gather(table, x.astype(jnp.int32))
